# Initial kernel scaffold; baseline (speedup 1.0000x reference)
import jax
import jax.numpy as jnp
from jax.experimental import pallas as pl


def kernel(hidden_states, Ws1, Ws2, Wg, We1, We2):
    raise NotImplementedError("write your pallas kernel here")


# speedup vs baseline: 1.1952x; 1.1952x over previous
import jax
import jax.numpy as jnp
from jax.experimental import pallas as pl
from jax.experimental.pallas import tpu as pltpu

HID = 1024
INTER = 4096
MINTER = 512
NE = 16

TT = 512
SH_IT = 1024
NSH = INTER // SH_IT
NSTEP = NSH + NE


def _moe_body(x_ref, wg_ref, ws1_ref, ws2_ref, we1_ref, we2_ref,
              out_ref, logits_ref, comb_ref):
    s = pl.program_id(1)

    @pl.when(s == 0)
    def _router():
        x = x_ref[...]
        logits = jax.lax.dot_general(
            x, wg_ref[...], (((1,), (1,)), ((), ())),
            preferred_element_type=jnp.float32)
        logits_ref[...] = logits
        lane = jax.lax.broadcasted_iota(jnp.int32, (TT, NE), 1)
        m1 = jnp.max(logits, axis=1, keepdims=True)
        i1 = jnp.min(jnp.where(logits >= m1, lane, NE), axis=1, keepdims=True)
        masked = jnp.where(lane == i1, -jnp.inf, logits)
        m2 = jnp.max(masked, axis=1, keepdims=True)
        i2 = jnp.min(jnp.where(masked >= m2, lane, NE), axis=1, keepdims=True)
        w1 = 1.0 / (1.0 + jnp.exp(m2 - m1))
        w2 = 1.0 - w1
        comb_ref[...] = (jnp.where(lane == i1, w1, 0.0)
                         + jnp.where(lane == i2, w2, 0.0))
        out_ref[...] = jnp.zeros_like(out_ref)

    @pl.when(s < NSH)
    def _shared():
        x = x_ref[...]
        h = jax.lax.dot_general(
            x, ws1_ref[...], (((1,), (1,)), ((), ())),
            preferred_element_type=jnp.float32)
        h = h * jax.nn.sigmoid(h)
        out_ref[...] += jax.lax.dot_general(
            h, ws2_ref[...], (((1,), (1,)), ((), ())),
            preferred_element_type=jnp.float32)

    @pl.when(s >= NSH)
    def _expert():
        e = s - NSH
        x = x_ref[...]
        h = jax.lax.dot_general(
            x, we1_ref[0], (((1,), (1,)), ((), ())),
            preferred_element_type=jnp.float32)
        h = h * jax.nn.sigmoid(h)
        lane = jax.lax.broadcasted_iota(jnp.int32, (TT, NE), 1)
        c = jnp.sum(jnp.where(lane == e, comb_ref[...], 0.0),
                    axis=1, keepdims=True)
        h = h * c
        out_ref[...] += jax.lax.dot_general(
            h, we2_ref[0], (((1,), (1,)), ((), ())),
            preferred_element_type=jnp.float32)


def kernel(hidden_states, Ws1, Ws2, Wg, We1, We2):
    B, S, H = hidden_states.shape
    T = B * S
    x = hidden_states.reshape(T, H)
    grid = (T // TT, NSTEP)

    out, logits = pl.pallas_call(
        _moe_body,
        grid=grid,
        in_specs=[
            pl.BlockSpec((TT, HID), lambda t, s: (t, 0)),
            pl.BlockSpec((NE, HID), lambda t, s: (0, 0)),
            pl.BlockSpec((SH_IT, HID), lambda t, s: (jnp.minimum(s, NSH - 1), 0)),
            pl.BlockSpec((HID, SH_IT), lambda t, s: (0, jnp.minimum(s, NSH - 1))),
            pl.BlockSpec((1, MINTER, HID), lambda t, s: (jnp.maximum(s - NSH, 0), 0, 0)),
            pl.BlockSpec((1, HID, MINTER), lambda t, s: (jnp.maximum(s - NSH, 0), 0, 0)),
        ],
        out_specs=[
            pl.BlockSpec((TT, HID), lambda t, s: (t, 0)),
            pl.BlockSpec((TT, NE), lambda t, s: (t, 0)),
        ],
        out_shape=[
            jax.ShapeDtypeStruct((T, HID), jnp.float32),
            jax.ShapeDtypeStruct((T, NE), jnp.float32),
        ],
        scratch_shapes=[pltpu.VMEM((TT, NE), jnp.float32)],
        compiler_params=pltpu.CompilerParams(
            dimension_semantics=("parallel", "arbitrary")),
    )(x, Wg, Ws1, Ws2, We1, We2)

    return out.reshape(B, S, H), logits.reshape(B, S, NE)
